# baseline (device time: 53333 ns/iter reference)
import jax
import jax.numpy as jnp
from jax import lax
from jax.experimental import pallas as pl
from jax.experimental.pallas import tpu as pltpu

N_DEV = 32
N_LAYERS = 3
GENS = (1, 3, 4, 8, 16)
N_STAGES = len(GENS)
N_SLOTS = N_LAYERS * N_STAGES


def kernel(x, Win0, Wout0, Win1, Wout1, Win2, Wout2):
    b, d = x.shape

    def body(x_ref, win0_ref, wout0_ref, win1_ref, wout1_ref, win2_ref,
             wout2_ref, out_ref, comm_ref, send_sems, recv_sems):
        my = lax.axis_index("i")

        barrier_sem = pltpu.get_barrier_semaphore()
        for g in GENS:
            pl.semaphore_signal(
                barrier_sem, inc=1,
                device_id=(my ^ g,), device_id_type=pl.DeviceIdType.MESH,
            )
        pl.semaphore_wait(barrier_sem, N_STAGES)

        wins = (win0_ref, win1_ref, win2_ref)
        wouts = (wout0_ref, wout1_ref, wout2_ref)

        acc = x_ref[...]
        for layer in range(N_LAYERS):
            h = jnp.dot(
                acc.astype(jnp.bfloat16),
                wins[layer][...].astype(jnp.bfloat16),
                preferred_element_type=jnp.float32,
            )
            h = jnp.maximum(h, 0.0)
            acc = jnp.dot(
                h.astype(jnp.bfloat16),
                wouts[layer][...].astype(jnp.bfloat16),
                preferred_element_type=jnp.float32,
            )
            for k, g in enumerate(GENS):
                slot = layer * N_STAGES + k
                comm_ref[slot, 0] = acc
                rdma = pltpu.make_async_remote_copy(
                    src_ref=comm_ref.at[slot, 0],
                    dst_ref=comm_ref.at[slot, 1],
                    send_sem=send_sems.at[slot],
                    recv_sem=recv_sems.at[slot],
                    device_id=(my ^ g,),
                    device_id_type=pl.DeviceIdType.MESH,
                )
                rdma.start()
                rdma.wait()
                acc = acc + comm_ref[slot, 1]

        out_ref[...] = acc

    return pl.pallas_call(
        body,
        out_shape=jax.ShapeDtypeStruct((b, d), jnp.float32),
        in_specs=[pl.BlockSpec(memory_space=pltpu.VMEM)] * 7,
        out_specs=pl.BlockSpec(memory_space=pltpu.VMEM),
        scratch_shapes=[
            pltpu.VMEM((N_SLOTS, 2, b, d), jnp.float32),
            pltpu.SemaphoreType.DMA((N_SLOTS,)),
            pltpu.SemaphoreType.DMA((N_SLOTS,)),
        ],
        compiler_params=pltpu.CompilerParams(collective_id=0),
    )(x, Win0, Wout0, Win1, Wout1, Win2, Wout2)


# device time: 46594 ns/iter; 1.1446x vs baseline; 1.1446x over previous
import jax
import jax.numpy as jnp
from jax import lax
from jax.experimental import pallas as pl
from jax.experimental.pallas import tpu as pltpu

N_DEV = 32
N_LAYERS = 3
GENS = (1, 3, 4, 8, 16)
N_STAGES = len(GENS)
N_SLOTS = N_LAYERS * N_STAGES


def kernel(x, Win0, Wout0, Win1, Wout1, Win2, Wout2):
    b, d = x.shape

    def body(x_ref, win0_ref, wout0_ref, win1_ref, wout1_ref, win2_ref,
             wout2_ref, out_ref, comm_ref, send_sems, recv_sems):
        my = lax.axis_index("i")

        barrier_sem = pltpu.get_barrier_semaphore()
        for g in GENS:
            pl.semaphore_signal(
                barrier_sem, inc=1,
                device_id=(my ^ g,), device_id_type=pl.DeviceIdType.MESH,
            )
        pl.semaphore_wait(barrier_sem, N_STAGES)

        wins = (win0_ref, win1_ref, win2_ref)
        wouts = (wout0_ref, wout1_ref, wout2_ref)

        pending_sends = []
        acc = x_ref[...]
        for layer in range(N_LAYERS):
            h = jnp.dot(
                acc.astype(jnp.bfloat16),
                wins[layer][...].astype(jnp.bfloat16),
                preferred_element_type=jnp.float32,
            )
            h = jnp.maximum(h, 0.0)
            acc = jnp.dot(
                h.astype(jnp.bfloat16),
                wouts[layer][...].astype(jnp.bfloat16),
                preferred_element_type=jnp.float32,
            )
            for k, g in enumerate(GENS):
                slot = layer * N_STAGES + k
                comm_ref[slot, 0] = acc.astype(jnp.bfloat16)
                rdma = pltpu.make_async_remote_copy(
                    src_ref=comm_ref.at[slot, 0],
                    dst_ref=comm_ref.at[slot, 1],
                    send_sem=send_sems.at[slot],
                    recv_sem=recv_sems.at[slot],
                    device_id=(my ^ g,),
                    device_id_type=pl.DeviceIdType.MESH,
                )
                rdma.start()
                pending_sends.append(rdma)
                rdma.wait_recv()
                acc = acc + comm_ref[slot, 1].astype(jnp.float32)

        out_ref[...] = acc
        for rdma in pending_sends:
            rdma.wait_send()

    return pl.pallas_call(
        body,
        out_shape=jax.ShapeDtypeStruct((b, d), jnp.float32),
        in_specs=[pl.BlockSpec(memory_space=pltpu.VMEM)] * 7,
        out_specs=pl.BlockSpec(memory_space=pltpu.VMEM),
        scratch_shapes=[
            pltpu.VMEM((N_SLOTS, 2, b, d), jnp.bfloat16),
            pltpu.SemaphoreType.DMA((N_SLOTS,)),
            pltpu.SemaphoreType.DMA((N_SLOTS,)),
        ],
        compiler_params=pltpu.CompilerParams(collective_id=0),
    )(x, Win0, Wout0, Win1, Wout1, Win2, Wout2)


# device time: 41692 ns/iter; 1.2792x vs baseline; 1.1176x over previous
import jax
import jax.numpy as jnp
from jax import lax
from jax.experimental import pallas as pl
from jax.experimental.pallas import tpu as pltpu

N_DEV = 32
N_LAYERS = 3
GROUPS = ((1, 2, 3), (4, 8, 12), (16,))
ALL_GENS = tuple(g for grp in GROUPS for g in grp)
N_STAGES = len(GROUPS)
N_RECV = len(ALL_GENS)
N_SLOTS = N_LAYERS * N_RECV


def kernel(x, Win0, Wout0, Win1, Wout1, Win2, Wout2):
    b, d = x.shape

    def body(x_ref, win0_ref, wout0_ref, win1_ref, wout1_ref, win2_ref,
             wout2_ref, out_ref, send_ref, recv_ref, send_sems, recv_sems):
        my = lax.axis_index("i")

        barrier_sem = pltpu.get_barrier_semaphore()
        for g in ALL_GENS:
            pl.semaphore_signal(
                barrier_sem, inc=1,
                device_id=(my ^ g,), device_id_type=pl.DeviceIdType.MESH,
            )
        pl.semaphore_wait(barrier_sem, N_RECV)

        wins = (win0_ref, win1_ref, win2_ref)
        wouts = (wout0_ref, wout1_ref, wout2_ref)

        pending_sends = []
        acc = x_ref[...]
        for layer in range(N_LAYERS):
            h = jnp.dot(
                acc.astype(jnp.bfloat16),
                wins[layer][...].astype(jnp.bfloat16),
                preferred_element_type=jnp.float32,
            )
            h = jnp.maximum(h, 0.0)
            acc = jnp.dot(
                h.astype(jnp.bfloat16),
                wouts[layer][...].astype(jnp.bfloat16),
                preferred_element_type=jnp.float32,
            )
            slot = layer * N_RECV
            for si, group in enumerate(GROUPS):
                send_slot = layer * N_STAGES + si
                send_ref[send_slot] = acc.astype(jnp.bfloat16)
                started = []
                for g in group:
                    rdma = pltpu.make_async_remote_copy(
                        src_ref=send_ref.at[send_slot],
                        dst_ref=recv_ref.at[slot],
                        send_sem=send_sems.at[slot],
                        recv_sem=recv_sems.at[slot],
                        device_id=(my ^ g,),
                        device_id_type=pl.DeviceIdType.MESH,
                    )
                    rdma.start()
                    started.append((rdma, slot))
                    slot += 1
                pending_sends.extend(r for r, _ in started)
                for rdma, rslot in started:
                    rdma.wait_recv()
                    acc = acc + recv_ref[rslot].astype(jnp.float32)

        out_ref[...] = acc
        for rdma in pending_sends:
            rdma.wait_send()

    return pl.pallas_call(
        body,
        out_shape=jax.ShapeDtypeStruct((b, d), jnp.float32),
        in_specs=[pl.BlockSpec(memory_space=pltpu.VMEM)] * 7,
        out_specs=pl.BlockSpec(memory_space=pltpu.VMEM),
        scratch_shapes=[
            pltpu.VMEM((N_LAYERS * N_STAGES, b, d), jnp.bfloat16),
            pltpu.VMEM((N_SLOTS, b, d), jnp.bfloat16),
            pltpu.SemaphoreType.DMA((N_SLOTS,)),
            pltpu.SemaphoreType.DMA((N_SLOTS,)),
        ],
        compiler_params=pltpu.CompilerParams(collective_id=0),
    )(x, Win0, Wout0, Win1, Wout1, Win2, Wout2)


# device time: 40562 ns/iter; 1.3149x vs baseline; 1.0279x over previous
import jax
import jax.numpy as jnp
from jax import lax
from jax.experimental import pallas as pl
from jax.experimental.pallas import tpu as pltpu

N_DEV = 32
N_LAYERS = 3
GROUPS = ((1, 3, 2, 4, 5, 7, 6), (8, 16, 24))
ALL_GENS = tuple(g for grp in GROUPS for g in grp)
N_STAGES = len(GROUPS)
N_RECV = len(ALL_GENS)
N_SLOTS = N_LAYERS * N_RECV


def kernel(x, Win0, Wout0, Win1, Wout1, Win2, Wout2):
    b, d = x.shape

    def body(x_ref, win0_ref, wout0_ref, win1_ref, wout1_ref, win2_ref,
             wout2_ref, out_ref, send_ref, recv_ref, send_sems, recv_sems):
        my = lax.axis_index("i")

        barrier_sem = pltpu.get_barrier_semaphore()
        for g in ALL_GENS:
            pl.semaphore_signal(
                barrier_sem, inc=1,
                device_id=(my ^ g,), device_id_type=pl.DeviceIdType.MESH,
            )
        pl.semaphore_wait(barrier_sem, N_RECV)

        wins = (win0_ref, win1_ref, win2_ref)
        wouts = (wout0_ref, wout1_ref, wout2_ref)

        pending_sends = []
        acc = x_ref[...]
        for layer in range(N_LAYERS):
            h = jnp.dot(
                acc.astype(jnp.bfloat16),
                wins[layer][...].astype(jnp.bfloat16),
                preferred_element_type=jnp.float32,
            )
            h = jnp.maximum(h, 0.0)
            acc = jnp.dot(
                h.astype(jnp.bfloat16),
                wouts[layer][...].astype(jnp.bfloat16),
                preferred_element_type=jnp.float32,
            )
            slot = layer * N_RECV
            for si, group in enumerate(GROUPS):
                send_slot = layer * N_STAGES + si
                send_ref[send_slot] = acc.astype(jnp.bfloat16)
                started = []
                for g in group:
                    rdma = pltpu.make_async_remote_copy(
                        src_ref=send_ref.at[send_slot],
                        dst_ref=recv_ref.at[slot],
                        send_sem=send_sems.at[slot],
                        recv_sem=recv_sems.at[slot],
                        device_id=(my ^ g,),
                        device_id_type=pl.DeviceIdType.MESH,
                    )
                    rdma.start()
                    started.append((rdma, slot))
                    slot += 1
                pending_sends.extend(r for r, _ in started)
                for rdma, rslot in started:
                    rdma.wait_recv()
                    acc = acc + recv_ref[rslot].astype(jnp.float32)

        out_ref[...] = acc
        for rdma in pending_sends:
            rdma.wait_send()

    return pl.pallas_call(
        body,
        out_shape=jax.ShapeDtypeStruct((b, d), jnp.float32),
        in_specs=[pl.BlockSpec(memory_space=pltpu.VMEM)] * 7,
        out_specs=pl.BlockSpec(memory_space=pltpu.VMEM),
        scratch_shapes=[
            pltpu.VMEM((N_LAYERS * N_STAGES, b, d), jnp.bfloat16),
            pltpu.VMEM((N_SLOTS, b, d), jnp.bfloat16),
            pltpu.SemaphoreType.DMA((N_SLOTS,)),
            pltpu.SemaphoreType.DMA((N_SLOTS,)),
        ],
        compiler_params=pltpu.CompilerParams(collective_id=0),
    )(x, Win0, Wout0, Win1, Wout1, Win2, Wout2)


# device time: 39207 ns/iter; 1.3603x vs baseline; 1.0346x over previous
import jax
import jax.numpy as jnp
from jax import lax
from jax.experimental import pallas as pl
from jax.experimental.pallas import tpu as pltpu

N_DEV = 32
N_LAYERS = 3
GROUPS = ((1, 3, 2, 4, 5, 7, 6), (8, 16, 24))
ALL_GENS = tuple(g for grp in GROUPS for g in grp)
N_STAGES = len(GROUPS)
N_RECV = len(ALL_GENS)
N_SLOTS = N_LAYERS * N_RECV


def kernel(x, Win0, Wout0, Win1, Wout1, Win2, Wout2):
    b, d = x.shape

    def body(x_ref, win0_ref, wout0_ref, win1_ref, wout1_ref, win2_ref,
             wout2_ref, out_ref, send_ref, recv_ref, send_sems, recv_sems):
        my = lax.axis_index("i")

        barrier_sem = pltpu.get_barrier_semaphore()
        for g in ALL_GENS:
            pl.semaphore_signal(
                barrier_sem, inc=1,
                device_id=(my ^ g,), device_id_type=pl.DeviceIdType.MESH,
            )
        pl.semaphore_wait(barrier_sem, N_RECV)

        wins = (win0_ref, win1_ref, win2_ref)
        wouts = (wout0_ref, wout1_ref, wout2_ref)

        pending_sends = []
        acc = x_ref[...]
        for layer in range(N_LAYERS):
            h = jnp.dot(acc, wins[layer][...],
                        preferred_element_type=jnp.float32)
            h = jnp.maximum(h, 0.0).astype(jnp.bfloat16)
            acc = jnp.dot(h, wouts[layer][...],
                          preferred_element_type=jnp.float32
                          ).astype(jnp.bfloat16)
            slot = layer * N_RECV
            for si, group in enumerate(GROUPS):
                send_slot = layer * N_STAGES + si
                send_ref[send_slot] = acc
                started = []
                for g in group:
                    rdma = pltpu.make_async_remote_copy(
                        src_ref=send_ref.at[send_slot],
                        dst_ref=recv_ref.at[slot],
                        send_sem=send_sems.at[slot],
                        recv_sem=recv_sems.at[slot],
                        device_id=(my ^ g,),
                        device_id_type=pl.DeviceIdType.MESH,
                    )
                    rdma.start()
                    started.append((rdma, slot))
                    slot += 1
                pending_sends.extend(r for r, _ in started)
                for rdma, rslot in started:
                    rdma.wait_recv()
                    acc = acc + recv_ref[rslot]

        out_ref[...] = acc.astype(jnp.float32)
        for rdma in pending_sends:
            rdma.wait_send()

    bf = jnp.bfloat16
    return pl.pallas_call(
        body,
        out_shape=jax.ShapeDtypeStruct((b, d), jnp.float32),
        in_specs=[pl.BlockSpec(memory_space=pltpu.VMEM)] * 7,
        out_specs=pl.BlockSpec(memory_space=pltpu.VMEM),
        scratch_shapes=[
            pltpu.VMEM((N_LAYERS * N_STAGES, b, d), bf),
            pltpu.VMEM((N_SLOTS, b, d), bf),
            pltpu.SemaphoreType.DMA((N_SLOTS,)),
            pltpu.SemaphoreType.DMA((N_SLOTS,)),
        ],
        compiler_params=pltpu.CompilerParams(collective_id=0),
    )(x.astype(bf), Win0.astype(bf), Wout0.astype(bf), Win1.astype(bf),
      Wout1.astype(bf), Win2.astype(bf), Wout2.astype(bf))
